# baseline (speedup 1.0000x reference)
import functools

import jax
import jax.numpy as jnp
from jax.experimental import pallas as pl
from jax.experimental.pallas import tpu as pltpu

_BETA = 0.25
_NZ = 256
_TM = 1024


def _dot_t(h, w):
    return jax.lax.dot_general(h, w.astype(jnp.bfloat16),
                               (((1,), (1,)), ((), ())),
                               preferred_element_type=jnp.float32)


def _lin(h, w_ref, b_ref):
    return _dot_t(h, w_ref[...]) + b_ref[...]


def _rlin(h, w_ref, b_ref):
    return jnp.maximum(_lin(h, w_ref, b_ref), 0.0).astype(jnp.bfloat16)


def _enc_body(xm_ref, xa_ref,
              wim, bim, wia, bia,
              em1, eb1, em2, eb2, em3, eb3,
              ea1, ab1, ea2, ab2, ea3, ab3,
              es1, sb1, es2, sb2, es3, sb3,
              cb,
              o_zmp, o_zms, o_zap, o_zas, o_loss,
              *, loss_scale):
    bf16 = jnp.bfloat16
    f32 = jnp.float32
    tm = xm_ref.shape[0]

    hm = _rlin(xm_ref[...].astype(bf16), wim, bim)
    ha = _rlin(xa_ref[...].astype(bf16), wia, bia)

    zmp = _lin(_rlin(_rlin(hm, em1, eb1), em2, eb2), em3, eb3)
    zap = _lin(_rlin(_rlin(ha, ea1, ab1), ea2, ab2), ea3, ab3)

    hs = jnp.concatenate([hm, ha], axis=0)
    zs = _lin(_rlin(_rlin(hs, es1, sb1), es2, sb2), es3, sb3)
    zms = zs[:tm]
    zas = zs[tm:]

    o_zmp[...] = zmp
    o_zms[...] = zms
    o_zap[...] = zap
    o_zas[...] = zas

    cb32 = cb[...]
    c2 = jnp.sum(cb32 * cb32, axis=1, keepdims=True).T
    s = jax.lax.dot_general(zs.astype(bf16), cb32.astype(bf16),
                            (((1,), (1,)), ((), ())),
                            preferred_element_type=f32)
    z2 = jnp.sum(zs * zs, axis=1, keepdims=True)
    tile_loss = jnp.sum(jnp.max(c2 - 2.0 * s, axis=1, keepdims=True) + z2)

    @pl.when(pl.program_id(0) == 0)
    def _():
        o_loss[...] = jnp.zeros_like(o_loss)

    o_loss[...] += jnp.full((1, 1), tile_loss * loss_scale, dtype=f32)


def _dec_body(zmp_ref, zms_ref, zap_ref, zas_ref,
              dw1, db1, dw2, db2, dw3, db3,
              wdm, bdm, wda, bda,
              o_xmh, o_xah):
    bf16 = jnp.bfloat16
    tm = zmp_ref.shape[0]

    zd = jnp.concatenate([zmp_ref[...] + zms_ref[...],
                          zap_ref[...] + zas_ref[...]], axis=0).astype(bf16)
    hd = _rlin(_rlin(_rlin(zd, dw1, db1), dw2, db2), dw3, db3)
    o_xmh[...] = _lin(hd[:tm], wdm, bdm)
    o_xah[...] = _lin(hd[tm:], wda, bda)


def kernel(x_m, x_a, params):
    p = params
    B = x_m.shape[0]
    f32 = jnp.float32

    def bb(name):
        return p[name].reshape(1, -1)

    n_in_m = x_m.shape[1]
    n_in_a = x_a.shape[1]
    grid = (B // _TM,)

    def tile_spec(cols):
        return pl.BlockSpec((_TM, cols), lambda i: (i, 0))

    def full_spec(arr):
        return pl.BlockSpec(arr.shape, lambda i: (0,) * arr.ndim)

    cparams = pltpu.CompilerParams(
        dimension_semantics=("arbitrary",),
        vmem_limit_bytes=100 * 1024 * 1024,
    )

    enc_weights = [
        p['W_im'], bb('b_im'), p['W_ia'], bb('b_ia'),
        p['em_W1'], bb('em_b1'), p['em_W2'], bb('em_b2'), p['em_W3'], bb('em_b3'),
        p['ea_W1'], bb('ea_b1'), p['ea_W2'], bb('ea_b2'), p['ea_W3'], bb('ea_b3'),
        p['es_W1'], bb('es_b1'), p['es_W2'], bb('es_b2'), p['es_W3'], bb('es_b3'),
        p['codebook'],
    ]
    enc_in_specs = [tile_spec(n_in_m), tile_spec(n_in_a)]
    enc_in_specs += [full_spec(a) for a in enc_weights]
    enc_out_shape = [
        jax.ShapeDtypeStruct((B, _NZ), f32),
        jax.ShapeDtypeStruct((B, _NZ), f32),
        jax.ShapeDtypeStruct((B, _NZ), f32),
        jax.ShapeDtypeStruct((B, _NZ), f32),
        jax.ShapeDtypeStruct((1, 1), f32),
    ]
    enc_out_specs = [
        tile_spec(_NZ), tile_spec(_NZ), tile_spec(_NZ), tile_spec(_NZ),
        pl.BlockSpec((1, 1), lambda i: (0, 0)),
    ]
    loss_scale = (1.0 + _BETA) / (B * _NZ)
    enc_fn = functools.partial(_enc_body, loss_scale=loss_scale)
    zmp, zms, zap, zas, loss = pl.pallas_call(
        enc_fn,
        grid=grid,
        in_specs=enc_in_specs,
        out_specs=enc_out_specs,
        out_shape=enc_out_shape,
        compiler_params=cparams,
    )(x_m, x_a, *enc_weights)

    dec_weights = [
        p['d_W1'], bb('d_b1'), p['d_W2'], bb('d_b2'), p['d_W3'], bb('d_b3'),
        p['W_dm'], bb('b_dm'), p['W_da'], bb('b_da'),
    ]
    dec_in_specs = [tile_spec(_NZ)] * 4 + [full_spec(a) for a in dec_weights]
    dec_out_shape = [
        jax.ShapeDtypeStruct((B, n_in_m), f32),
        jax.ShapeDtypeStruct((B, n_in_a), f32),
    ]
    dec_out_specs = [tile_spec(n_in_m), tile_spec(n_in_a)]
    xmh, xah = pl.pallas_call(
        _dec_body,
        grid=grid,
        in_specs=dec_in_specs,
        out_specs=dec_out_specs,
        out_shape=dec_out_shape,
        compiler_params=cparams,
    )(zmp, zms, zap, zas, *dec_weights)

    return ((zmp, zms, zap, zas), (xmh, xah), loss.reshape(()))


# speedup vs baseline: 1.0222x; 1.0222x over previous
import functools

import jax
import jax.numpy as jnp
from jax.experimental import pallas as pl
from jax.experimental.pallas import tpu as pltpu

_BETA = 0.25
_NZ = 256
_TM = 512


def _body(xm_ref, xa_ref,
          wim, bim, wia, bia,
          em1, eb1, em2, eb2, em3, eb3,
          ea1, ab1, ea2, ab2, ea3, ab3,
          es1, sb1, es2, sb2, es3, sb3,
          cb,
          dw1, db1, dw2, db2, dw3, db3,
          wdm, bdm, wda, bda,
          o_zmp, o_zms, o_zap, o_zas, o_xmh, o_xah, o_loss,
          *, loss_scale):
    bf16 = jnp.bfloat16
    f32 = jnp.float32

    def lin(h, w, b):
        y = jax.lax.dot_general(h, w[...].astype(bf16), (((1,), (1,)), ((), ())),
                                preferred_element_type=f32)
        return y + b[...]

    def rlin(h, w, b):
        return jnp.maximum(lin(h, w, b), 0.0).astype(bf16)

    tm = xm_ref.shape[0]
    hm = rlin(xm_ref[...].astype(bf16), wim, bim)
    ha = rlin(xa_ref[...].astype(bf16), wia, bia)

    zmp = lin(rlin(rlin(hm, em1, eb1), em2, eb2), em3, eb3)
    zap = lin(rlin(rlin(ha, ea1, ab1), ea2, ab2), ea3, ab3)

    zms = lin(rlin(rlin(hm, es1, sb1), es2, sb2), es3, sb3)
    zas = lin(rlin(rlin(ha, es1, sb1), es2, sb2), es3, sb3)

    o_zmp[...] = zmp
    o_zms[...] = zms
    o_zap[...] = zap
    o_zas[...] = zas

    cb32 = cb[...]
    c2 = jnp.sum(cb32 * cb32, axis=1, keepdims=True).T
    cbm2 = (-2.0 * cb32).astype(bf16)

    def qpart(z):
        sm2 = jax.lax.dot_general(z.astype(bf16), cbm2,
                                  (((1,), (1,)), ((), ())),
                                  preferred_element_type=f32)
        z2 = jnp.sum(z * z, axis=1, keepdims=True)
        return jnp.sum(jnp.max(c2 + sm2, axis=1, keepdims=True) + z2)

    tile_loss = qpart(zms) + qpart(zas)

    @pl.when(pl.program_id(0) == 0)
    def _():
        o_loss[...] = jnp.zeros_like(o_loss)

    o_loss[...] += jnp.full((1, 1), tile_loss * loss_scale, dtype=f32)

    hdm = rlin(rlin(rlin((zmp + zms).astype(bf16), dw1, db1), dw2, db2), dw3, db3)
    hda = rlin(rlin(rlin((zap + zas).astype(bf16), dw1, db1), dw2, db2), dw3, db3)
    o_xmh[...] = lin(hdm, wdm, bdm)
    o_xah[...] = lin(hda, wda, bda)


def kernel(x_m, x_a, params):
    p = params
    B = x_m.shape[0]
    f32 = jnp.float32

    def bb(name):
        return p[name].reshape(1, -1)

    weight_args = [
        p['W_im'], bb('b_im'), p['W_ia'], bb('b_ia'),
        p['em_W1'], bb('em_b1'), p['em_W2'], bb('em_b2'), p['em_W3'], bb('em_b3'),
        p['ea_W1'], bb('ea_b1'), p['ea_W2'], bb('ea_b2'), p['ea_W3'], bb('ea_b3'),
        p['es_W1'], bb('es_b1'), p['es_W2'], bb('es_b2'), p['es_W3'], bb('es_b3'),
        p['codebook'],
        p['d_W1'], bb('d_b1'), p['d_W2'], bb('d_b2'), p['d_W3'], bb('d_b3'),
        p['W_dm'], bb('b_dm'), p['W_da'], bb('b_da'),
    ]

    n_in_m = x_m.shape[1]
    n_in_a = x_a.shape[1]
    grid = (B // _TM,)

    def tile_spec(cols):
        return pl.BlockSpec((_TM, cols), lambda i: (i, 0))

    def full_spec(arr):
        return pl.BlockSpec(arr.shape, lambda i: (0,) * arr.ndim)

    in_specs = [tile_spec(n_in_m), tile_spec(n_in_a)]
    in_specs += [full_spec(a) for a in weight_args]

    out_shape = [
        jax.ShapeDtypeStruct((B, _NZ), f32),
        jax.ShapeDtypeStruct((B, _NZ), f32),
        jax.ShapeDtypeStruct((B, _NZ), f32),
        jax.ShapeDtypeStruct((B, _NZ), f32),
        jax.ShapeDtypeStruct((B, n_in_m), f32),
        jax.ShapeDtypeStruct((B, n_in_a), f32),
        jax.ShapeDtypeStruct((1, 1), f32),
    ]
    out_specs = [
        tile_spec(_NZ), tile_spec(_NZ), tile_spec(_NZ), tile_spec(_NZ),
        tile_spec(n_in_m), tile_spec(n_in_a),
        pl.BlockSpec((1, 1), lambda i: (0, 0)),
    ]

    loss_scale = (1.0 + _BETA) / (B * _NZ)
    body = functools.partial(_body, loss_scale=loss_scale)

    zmp, zms, zap, zas, xmh, xah, loss = pl.pallas_call(
        body,
        grid=grid,
        in_specs=in_specs,
        out_specs=out_specs,
        out_shape=out_shape,
        compiler_params=pltpu.CompilerParams(
            dimension_semantics=("arbitrary",),
            vmem_limit_bytes=100 * 1024 * 1024,
        ),
    )(x_m, x_a, *weight_args)

    return ((zmp, zms, zap, zas), (xmh, xah), loss.reshape(()))


# speedup vs baseline: 1.0352x; 1.0128x over previous
import functools

import jax
import jax.numpy as jnp
from jax.experimental import pallas as pl
from jax.experimental.pallas import tpu as pltpu

_BETA = 0.25
_NZ = 256
_TM = 512


def _body(xm_ref, xa_ref,
          wim, bim, wia, bia,
          em1, eb1, em2, eb2, em3, eb3,
          ea1, ab1, ea2, ab2, ea3, ab3,
          es1, sb1, es2, sb2, es3, sb3,
          cb,
          dw1, db1, dw2, db2, dw3, db3,
          wdm, bdm, wda, bda,
          o_zmp, o_zms, o_zap, o_zas, o_xmh, o_xah, o_loss,
          *, loss_scale):
    bf16 = jnp.bfloat16
    f32 = jnp.float32

    def lin(h, w, b):
        y = jax.lax.dot_general(h, w[...].astype(bf16), (((1,), (1,)), ((), ())),
                                preferred_element_type=f32)
        return y + b[...]

    def rlin(h, w, b):
        return jnp.maximum(lin(h, w, b), 0.0).astype(bf16)

    tm = xm_ref.shape[0]
    hm = rlin(xm_ref[...].astype(bf16), wim, bim)
    ha = rlin(xa_ref[...].astype(bf16), wia, bia)

    zmp = lin(rlin(rlin(hm, em1, eb1), em2, eb2), em3, eb3)
    zap = lin(rlin(rlin(ha, ea1, ab1), ea2, ab2), ea3, ab3)

    zms = lin(rlin(rlin(hm, es1, sb1), es2, sb2), es3, sb3)
    zas = lin(rlin(rlin(ha, es1, sb1), es2, sb2), es3, sb3)

    o_zmp[...] = zmp
    o_zms[...] = zms
    o_zap[...] = zap
    o_zas[...] = zas

    hdm = rlin(rlin(rlin((zmp + zms).astype(bf16), dw1, db1), dw2, db2), dw3, db3)
    hda = rlin(rlin(rlin((zap + zas).astype(bf16), dw1, db1), dw2, db2), dw3, db3)
    o_xmh[...] = lin(hdm, wdm, bdm)
    o_xah[...] = lin(hda, wda, bda)

    cb32 = cb[...]
    c2 = jnp.sum(cb32 * cb32, axis=1, keepdims=True).T
    cbm2 = (-2.0 * cb32).astype(bf16)

    def qpart(z):
        sm2 = jax.lax.dot_general(z.astype(bf16), cbm2,
                                  (((1,), (1,)), ((), ())),
                                  preferred_element_type=f32)
        z2 = jnp.sum(z * z, axis=1, keepdims=True)
        return jnp.sum(jnp.max(c2 + sm2, axis=1, keepdims=True) + z2)

    tile_loss = qpart(zms) + qpart(zas)

    @pl.when(pl.program_id(0) == 0)
    def _():
        o_loss[...] = jnp.zeros_like(o_loss)

    o_loss[...] += jnp.full((1, 1), tile_loss * loss_scale, dtype=f32)


def kernel(x_m, x_a, params):
    p = params
    B = x_m.shape[0]
    f32 = jnp.float32

    def bb(name):
        return p[name].reshape(1, -1)

    weight_args = [
        p['W_im'], bb('b_im'), p['W_ia'], bb('b_ia'),
        p['em_W1'], bb('em_b1'), p['em_W2'], bb('em_b2'), p['em_W3'], bb('em_b3'),
        p['ea_W1'], bb('ea_b1'), p['ea_W2'], bb('ea_b2'), p['ea_W3'], bb('ea_b3'),
        p['es_W1'], bb('es_b1'), p['es_W2'], bb('es_b2'), p['es_W3'], bb('es_b3'),
        p['codebook'],
        p['d_W1'], bb('d_b1'), p['d_W2'], bb('d_b2'), p['d_W3'], bb('d_b3'),
        p['W_dm'], bb('b_dm'), p['W_da'], bb('b_da'),
    ]

    n_in_m = x_m.shape[1]
    n_in_a = x_a.shape[1]
    grid = (B // _TM,)

    def tile_spec(cols):
        return pl.BlockSpec((_TM, cols), lambda i: (i, 0))

    def full_spec(arr):
        return pl.BlockSpec(arr.shape, lambda i: (0,) * arr.ndim)

    in_specs = [tile_spec(n_in_m), tile_spec(n_in_a)]
    in_specs += [full_spec(a) for a in weight_args]

    out_shape = [
        jax.ShapeDtypeStruct((B, _NZ), f32),
        jax.ShapeDtypeStruct((B, _NZ), f32),
        jax.ShapeDtypeStruct((B, _NZ), f32),
        jax.ShapeDtypeStruct((B, _NZ), f32),
        jax.ShapeDtypeStruct((B, n_in_m), f32),
        jax.ShapeDtypeStruct((B, n_in_a), f32),
        jax.ShapeDtypeStruct((1, 1), f32),
    ]
    out_specs = [
        tile_spec(_NZ), tile_spec(_NZ), tile_spec(_NZ), tile_spec(_NZ),
        tile_spec(n_in_m), tile_spec(n_in_a),
        pl.BlockSpec((1, 1), lambda i: (0, 0)),
    ]

    loss_scale = (1.0 + _BETA) / (B * _NZ)
    body = functools.partial(_body, loss_scale=loss_scale)

    zmp, zms, zap, zas, xmh, xah, loss = pl.pallas_call(
        body,
        grid=grid,
        in_specs=in_specs,
        out_specs=out_specs,
        out_shape=out_shape,
        compiler_params=pltpu.CompilerParams(
            dimension_semantics=("arbitrary",),
            vmem_limit_bytes=100 * 1024 * 1024,
        ),
    )(x_m, x_a, *weight_args)

    return ((zmp, zms, zap, zas), (xmh, xah), loss.reshape(()))
